# initial kernel scaffold (unmeasured)
import functools

import jax
import jax.numpy as jnp
from jax import lax
from jax.experimental import pallas as pl
from jax.experimental.pallas import tpu as pltpu

T = 1024
D = 2048
V_LOCAL = 16384
CHUNK = 2048
N_CHUNKS = V_LOCAL // CHUNK
NEG_BIG = -1e30


def _stats_body(x_ref, w_ref, l_ref, m_out, s_out, ll_out, x_bf):
    j = pl.program_id(0)

    @pl.when(j == 0)
    def _():
        x_bf[...] = x_ref[...].astype(jnp.bfloat16)
        m_out[...] = jnp.full((T, 1), NEG_BIG, jnp.float32)
        s_out[...] = jnp.zeros((T, 1), jnp.float32)
        ll_out[...] = jnp.zeros((T, 1), jnp.float32)

    w_bf = w_ref[...].astype(jnp.bfloat16)
    logits = lax.dot_general(
        x_bf[...], w_bf, (((1,), (0,)), ((), ())),
        preferred_element_type=jnp.float32,
    )

    m_old = m_out[...]
    m_new = jnp.maximum(m_old, jnp.max(logits, axis=1, keepdims=True))
    s_out[...] = s_out[...] * jnp.exp(m_old - m_new) + jnp.sum(
        jnp.exp(logits - m_new), axis=1, keepdims=True
    )
    m_out[...] = m_new

    my_y = lax.axis_index("y")
    col0 = my_y * V_LOCAL + j * CHUNK
    cols = lax.broadcasted_iota(jnp.int32, (T, CHUNK), 1) + col0
    hit = cols == l_ref[...]
    ll_out[...] += jnp.sum(jnp.where(hit, logits, 0.0), axis=1, keepdims=True)


def _combine_body(m_ref, s_ref, ll_ref, out_ref, rm, rs, rll,
                  send_sems, recv_sems):
    my_x = lax.axis_index("x")
    my_y = lax.axis_index("y")
    my_z = lax.axis_index("z")
    nbr = (my_x, 1 - my_y, my_z)

    barrier = pltpu.get_barrier_semaphore()
    pl.semaphore_signal(barrier, inc=1, device_id=nbr,
                        device_id_type=pl.DeviceIdType.MESH)
    pl.semaphore_wait(barrier, 1)

    copies = []
    for k, (src, dst) in enumerate(((m_ref, rm), (s_ref, rs), (ll_ref, rll))):
        c = pltpu.make_async_remote_copy(
            src_ref=src, dst_ref=dst,
            send_sem=send_sems.at[k], recv_sem=recv_sems.at[k],
            device_id=nbr, device_id_type=pl.DeviceIdType.MESH,
        )
        c.start()
        copies.append(c)
    for c in copies:
        c.wait()

    m_o = m_ref[...]
    m_r = rm[...]
    m_n = jnp.maximum(m_o, m_r)
    s_n = s_ref[...] * jnp.exp(m_o - m_n) + rs[...] * jnp.exp(m_r - m_n)
    out_ref[...] = m_n + jnp.log(s_n) - (ll_ref[...] + rll[...])

    @functools.partial(pl.run_scoped, sem2=pltpu.SemaphoreType.REGULAR)
    def _(sem2):
        pl.semaphore_signal(sem2, inc=1, device_id=nbr,
                            device_id_type=pl.DeviceIdType.MESH)
        pl.semaphore_wait(sem2, 1)


def kernel(x, W, labels):
    labels2d = labels.reshape(T, 1)

    m, s, ll = pl.pallas_call(
        _stats_body,
        grid=(N_CHUNKS,),
        in_specs=[
            pl.BlockSpec((T, D), lambda j: (0, 0)),
            pl.BlockSpec((D, CHUNK), lambda j: (0, j)),
            pl.BlockSpec((T, 1), lambda j: (0, 0)),
        ],
        out_specs=[
            pl.BlockSpec((T, 1), lambda j: (0, 0)),
            pl.BlockSpec((T, 1), lambda j: (0, 0)),
            pl.BlockSpec((T, 1), lambda j: (0, 0)),
        ],
        out_shape=[
            jax.ShapeDtypeStruct((T, 1), jnp.float32),
            jax.ShapeDtypeStruct((T, 1), jnp.float32),
            jax.ShapeDtypeStruct((T, 1), jnp.float32),
        ],
        scratch_shapes=[pltpu.VMEM((T, D), jnp.bfloat16)],
        compiler_params=pltpu.CompilerParams(
            dimension_semantics=("arbitrary",),
        ),
    )(x, W, labels2d)

    nll = pl.pallas_call(
        _combine_body,
        in_specs=[pl.BlockSpec(memory_space=pltpu.VMEM)] * 3,
        out_specs=pl.BlockSpec(memory_space=pltpu.VMEM),
        out_shape=jax.ShapeDtypeStruct((T, 1), jnp.float32),
        scratch_shapes=[
            pltpu.VMEM((T, 1), jnp.float32),
            pltpu.VMEM((T, 1), jnp.float32),
            pltpu.VMEM((T, 1), jnp.float32),
            pltpu.SemaphoreType.DMA((3,)),
            pltpu.SemaphoreType.DMA((3,)),
        ],
        compiler_params=pltpu.CompilerParams(collective_id=0),
    )(m, s, ll)

    return nll.reshape(T)


# baseline (device time: 128363 ns/iter reference)
import functools

import jax
import jax.numpy as jnp
from jax import lax
from jax.experimental import pallas as pl
from jax.experimental.pallas import tpu as pltpu

T = 1024
D = 2048
V_LOCAL = 16384
CHUNK = 2048
N_CHUNKS = V_LOCAL // CHUNK
NEG_BIG = -1e30


def _stats_body(x_ref, w_ref, l_ref, m_out, s_out, ll_out, x_bf):
    j = pl.program_id(0)

    @pl.when(j == 0)
    def _():
        x_bf[...] = x_ref[...].astype(jnp.bfloat16)
        m_out[...] = jnp.full((T, 1), NEG_BIG, jnp.float32)
        s_out[...] = jnp.zeros((T, 1), jnp.float32)
        ll_out[...] = jnp.zeros((T, 1), jnp.float32)

    w_bf = w_ref[...].astype(jnp.bfloat16)
    logits = lax.dot_general(
        x_bf[...], w_bf, (((1,), (0,)), ((), ())),
        preferred_element_type=jnp.float32,
    )

    m_old = m_out[...]
    m_new = jnp.maximum(m_old, jnp.max(logits, axis=1, keepdims=True))
    s_out[...] = s_out[...] * jnp.exp(m_old - m_new) + jnp.sum(
        jnp.exp(logits - m_new), axis=1, keepdims=True
    )
    m_out[...] = m_new

    my_y = lax.axis_index("y")
    col0 = my_y * V_LOCAL + j * CHUNK
    cols = lax.broadcasted_iota(jnp.int32, (T, CHUNK), 1) + col0
    hit = cols == l_ref[...]
    ll_out[...] += jnp.sum(jnp.where(hit, logits, 0.0), axis=1, keepdims=True)


def _combine_body(m_ref, s_ref, ll_ref, out_ref, rm, rs, rll,
                  send_sems, recv_sems):
    my_x = lax.axis_index("x")
    my_y = lax.axis_index("y")
    my_z = lax.axis_index("z")
    nbr = (my_x, 1 - my_y, my_z)

    barrier = pltpu.get_barrier_semaphore()
    pl.semaphore_signal(barrier, inc=1, device_id=nbr,
                        device_id_type=pl.DeviceIdType.MESH)
    pl.semaphore_wait(barrier, 1)

    copies = []
    for k, (src, dst) in enumerate(((m_ref, rm), (s_ref, rs), (ll_ref, rll))):
        c = pltpu.make_async_remote_copy(
            src_ref=src, dst_ref=dst,
            send_sem=send_sems.at[k], recv_sem=recv_sems.at[k],
            device_id=nbr, device_id_type=pl.DeviceIdType.MESH,
        )
        c.start()
        copies.append(c)
    for c in copies:
        c.wait()

    m_o = m_ref[...]
    m_r = rm[...]
    m_n = jnp.maximum(m_o, m_r)
    s_n = s_ref[...] * jnp.exp(m_o - m_n) + rs[...] * jnp.exp(m_r - m_n)
    out_ref[...] = m_n + jnp.log(s_n) - (ll_ref[...] + rll[...])

    @functools.partial(pl.run_scoped, sem2=pltpu.SemaphoreType.REGULAR)
    def _(sem2):
        pl.semaphore_signal(sem2, inc=1, device_id=nbr,
                            device_id_type=pl.DeviceIdType.MESH)
        pl.semaphore_wait(sem2, 1)


def kernel(x, W, labels):
    labels2d = labels.reshape(T, 1)

    m, s, ll = pl.pallas_call(
        _stats_body,
        grid=(N_CHUNKS,),
        in_specs=[
            pl.BlockSpec((T, D), lambda j: (0, 0)),
            pl.BlockSpec((D, CHUNK), lambda j: (0, j)),
            pl.BlockSpec((T, 1), lambda j: (0, 0)),
        ],
        out_specs=[
            pl.BlockSpec((T, 1), lambda j: (0, 0)),
            pl.BlockSpec((T, 1), lambda j: (0, 0)),
            pl.BlockSpec((T, 1), lambda j: (0, 0)),
        ],
        out_shape=[
            jax.ShapeDtypeStruct((T, 1), jnp.float32),
            jax.ShapeDtypeStruct((T, 1), jnp.float32),
            jax.ShapeDtypeStruct((T, 1), jnp.float32),
        ],
        scratch_shapes=[pltpu.VMEM((T, D), jnp.bfloat16)],
        compiler_params=pltpu.CompilerParams(
            dimension_semantics=("arbitrary",),
            vmem_limit_bytes=100 * 1024 * 1024,
        ),
    )(x, W, labels2d)

    nll = pl.pallas_call(
        _combine_body,
        in_specs=[pl.BlockSpec(memory_space=pltpu.VMEM)] * 3,
        out_specs=pl.BlockSpec(memory_space=pltpu.VMEM),
        out_shape=jax.ShapeDtypeStruct((T, 1), jnp.float32),
        scratch_shapes=[
            pltpu.VMEM((T, 1), jnp.float32),
            pltpu.VMEM((T, 1), jnp.float32),
            pltpu.VMEM((T, 1), jnp.float32),
            pltpu.SemaphoreType.DMA((3,)),
            pltpu.SemaphoreType.DMA((3,)),
        ],
        compiler_params=pltpu.CompilerParams(collective_id=0),
    )(m, s, ll)

    return nll.reshape(T)
